# baseline (device time: 20442 ns/iter reference)
import jax
import jax.numpy as jnp
from jax import lax
from jax.experimental import pallas as pl
from jax.experimental.pallas import tpu as pltpu

N_DEV = 8
B = 2
SQ = 128
D = 512
H = 8
DH = 64
SKV = 128

_AXIS_MASKS = (1, 3, 4)

_BF16 = jnp.bfloat16


def _body(x_ref, wq_ref, wo_ref, k_ref, v_ref, out_ref,
          comm_ref, send_ref, send_sems, recv_sems):
    my = lax.axis_index("i")

    barrier = pltpu.get_barrier_semaphore()
    for mask in _AXIS_MASKS:
        pl.semaphore_signal(barrier, inc=1, device_id=(my ^ mask,),
                            device_id_type=pl.DeviceIdType.MESH)
    pl.semaphore_wait(barrier, len(_AXIS_MASKS))

    def exchange(r, half):
        mask = _AXIS_MASKS[(r + half) % 3]
        return pltpu.make_async_remote_copy(
            src_ref=send_ref.at[r, half],
            dst_ref=comm_ref.at[r, half],
            send_sem=send_sems.at[r, half],
            recv_sem=recv_sems.at[r, half],
            device_id=(my ^ mask,),
            device_id_type=pl.DeviceIdType.MESH,
        )

    wq16 = wq_ref[...].astype(_BF16)
    wo16 = wo_ref[...].astype(_BF16)
    rdmas0 = [None, None]
    for b in range(B):
        xb = x_ref[b].astype(_BF16)
        q = jnp.dot(xb, wq16, preferred_element_type=jnp.float32)
        q16 = q.astype(_BF16)
        k16 = k_ref[b].astype(_BF16)
        v16 = v_ref[b].astype(_BF16)
        scores = []
        for h in range(H):
            qh = q16[:, h * DH:(h + 1) * DH]
            kh = k16[:, h * DH:(h + 1) * DH]
            scores.append(lax.dot_general(
                qh, kh, (((1,), (1,)), ((), ())),
                preferred_element_type=jnp.float32))
        s3 = jnp.concatenate(scores, axis=-1).reshape(SQ, H, SKV) * 0.125
        m = jnp.max(s3, axis=-1, keepdims=True)
        p = jnp.exp(s3 - m)
        l = jnp.sum(p, axis=-1, keepdims=True)
        pn16 = (p / l).astype(_BF16).reshape(SQ, H * SKV)
        heads = []
        for h in range(H):
            ph = pn16[:, h * SKV:(h + 1) * SKV]
            vh = v16[:, h * DH:(h + 1) * DH]
            heads.append(jnp.dot(ph, vh, preferred_element_type=jnp.float32))
        ao = jnp.concatenate(heads, axis=-1).astype(_BF16)
        part = jnp.dot(ao, wo16, preferred_element_type=jnp.float32)
        send_ref[0, b] = part.astype(_BF16)
        rdmas0[b] = exchange(0, b)
        rdmas0[b].start()

    rdmas = rdmas0
    for r in range(3):
        nxt = [None, None]
        for half in (1, 0):
            rdmas[half].wait_recv()
            acc = send_ref[r, half] + comm_ref[r, half]
            if r < 2:
                send_ref[r + 1, half] = acc
                nxt[half] = exchange(r + 1, half)
                nxt[half].start()
            else:
                out_ref[half] = acc.astype(jnp.float32)
        rdmas = nxt

    for r in range(3):
        for half in range(B):
            exchange(r, half).wait_send()


def kernel(x, Wq, Wo, K_ext, V_ext):
    k2 = K_ext.reshape(B, SKV, H * DH)
    v2 = V_ext.reshape(B, SKV, H * DH)
    return pl.pallas_call(
        _body,
        out_shape=jax.ShapeDtypeStruct((B, SQ, D), jnp.float32),
        in_specs=[pl.BlockSpec(memory_space=pltpu.VMEM)] * 5,
        out_specs=pl.BlockSpec(memory_space=pltpu.VMEM),
        scratch_shapes=[
            pltpu.VMEM((3, B, SQ, D), _BF16),
            pltpu.VMEM((3, B, SQ, D), _BF16),
            pltpu.SemaphoreType.DMA((3, B)),
            pltpu.SemaphoreType.DMA((3, B)),
        ],
        compiler_params=pltpu.CompilerParams(collective_id=0),
    )(x, Wq, Wo, k2, v2)


# device time: 20348 ns/iter; 1.0046x vs baseline; 1.0046x over previous
import jax
import jax.numpy as jnp
from jax import lax
from jax.experimental import pallas as pl
from jax.experimental.pallas import tpu as pltpu

N_DEV = 8
B = 2
SQ = 128
D = 512
H = 8
DH = 64
SKV = 128

_AXIS_MASKS = (1, 3, 4)

_BF16 = jnp.bfloat16


def _body(x_ref, wq_ref, wo_ref, k_ref, v_ref, out_ref,
          comm_ref, send_ref, send_sems, recv_sems):
    my = lax.axis_index("i")

    barrier = pltpu.get_barrier_semaphore()
    for mask in _AXIS_MASKS:
        pl.semaphore_signal(barrier, inc=1, device_id=(my ^ mask,),
                            device_id_type=pl.DeviceIdType.MESH)
    pl.semaphore_wait(barrier, len(_AXIS_MASKS))

    def exchange(r, half):
        mask = _AXIS_MASKS[(r + half) % 3]
        return pltpu.make_async_remote_copy(
            src_ref=send_ref.at[r, half],
            dst_ref=comm_ref.at[r, half],
            send_sem=send_sems.at[r, half],
            recv_sem=recv_sems.at[r, half],
            device_id=(my ^ mask,),
            device_id_type=pl.DeviceIdType.MESH,
        )

    wq16 = wq_ref[...].astype(_BF16)
    wo16 = wo_ref[...].astype(_BF16)
    x16 = x_ref[...].astype(_BF16)
    q16 = jnp.dot(x16, wq16,
                  preferred_element_type=jnp.float32).astype(_BF16)
    k16 = k_ref[...].astype(_BF16)
    v16 = v_ref[...].astype(_BF16)
    scores = []
    for b in range(B):
        for h in range(H):
            qh = q16[b * SQ:(b + 1) * SQ, h * DH:(h + 1) * DH]
            kh = k16[b * SKV:(b + 1) * SKV, h * DH:(h + 1) * DH]
            scores.append(lax.dot_general(
                qh, kh, (((1,), (1,)), ((), ())),
                preferred_element_type=jnp.float32))
    s3 = jnp.concatenate(scores, axis=-1).reshape(SQ, B * H, SKV) * 0.125
    m = jnp.max(s3, axis=-1, keepdims=True)
    p = jnp.exp(s3 - m)
    l = jnp.sum(p, axis=-1, keepdims=True)
    pn16 = (p / l).astype(_BF16).reshape(SQ, B * H * SKV)
    aos = []
    for b in range(B):
        heads = []
        for h in range(H):
            i = b * H + h
            ph = pn16[:, i * SKV:(i + 1) * SKV]
            vh = v16[b * SKV:(b + 1) * SKV, h * DH:(h + 1) * DH]
            heads.append(jnp.dot(ph, vh, preferred_element_type=jnp.float32))
        aos.append(jnp.concatenate(heads, axis=-1))
    ao = jnp.concatenate(aos, axis=0).astype(_BF16)
    part = jnp.dot(ao, wo16, preferred_element_type=jnp.float32)
    rdmas = [None, None]
    for half in range(B):
        send_ref[0, half] = part[half * SQ:(half + 1) * SQ].astype(_BF16)
        rdmas[half] = exchange(0, half)
        rdmas[half].start()

    for r in range(3):
        nxt = [None, None]
        for half in range(B):
            rdmas[half].wait_recv()
            acc = send_ref[r, half] + comm_ref[r, half]
            if r < 2:
                send_ref[r + 1, half] = acc
                nxt[half] = exchange(r + 1, half)
                nxt[half].start()
            else:
                out_ref[half] = acc.astype(jnp.float32)
        rdmas = nxt

    for r in range(3):
        for half in range(B):
            exchange(r, half).wait_send()


def kernel(x, Wq, Wo, K_ext, V_ext):
    x2 = x.reshape(B * SQ, D)
    k2 = K_ext.reshape(B * SKV, H * DH)
    v2 = V_ext.reshape(B * SKV, H * DH)
    return pl.pallas_call(
        _body,
        out_shape=jax.ShapeDtypeStruct((B, SQ, D), jnp.float32),
        in_specs=[pl.BlockSpec(memory_space=pltpu.VMEM)] * 5,
        out_specs=pl.BlockSpec(memory_space=pltpu.VMEM),
        scratch_shapes=[
            pltpu.VMEM((3, B, SQ, D), _BF16),
            pltpu.VMEM((3, B, SQ, D), _BF16),
            pltpu.SemaphoreType.DMA((3, B)),
            pltpu.SemaphoreType.DMA((3, B)),
        ],
        compiler_params=pltpu.CompilerParams(collective_id=0),
    )(x2, Wq, Wo, k2, v2)
